# Initial kernel scaffold; baseline (speedup 1.0000x reference)
import jax
import jax.numpy as jnp
from jax.experimental import pallas as pl


def kernel(state, left, inputad, W1, b1, W2, b2, ln_g, ln_b, Wq, bq, Wk, bk, Wv, bv, Wo, bo):
    raise NotImplementedError("write your pallas kernel here")


# speedup vs baseline: 4.4045x; 4.4045x over previous
import functools

import jax
import jax.numpy as jnp
from jax import lax
from jax.experimental import pallas as pl
from jax.experimental.pallas import tpu as pltpu, tpu_sc as plsc

_B, _S, _D, _H = 4, 2048, 1024, 8
_DK = _D // _H
_BS = _B * _S
_SCALE = float(_DK) ** 0.5
_R = 256
_CH = 64


def _sc_gather_body(left_hbm, idx_hbm, out_hbm, idx_v, rows_v, sem):
    info = plsc.get_sparse_core_info()
    nc, ns, nl = info.num_cores, info.num_subcores, info.num_lanes
    rows_per = _BS // (nc * ns)
    wid = lax.axis_index("s") * nc + lax.axis_index("c")
    base_row = wid * rows_per
    b_off = (base_row // _S) * _S
    off_vec = jnp.full((nl,), 1, jnp.int32) * b_off
    for c in range(rows_per // _CH):
        start = base_row + c * _CH
        pltpu.sync_copy(idx_hbm.at[pl.ds(start, _CH)], idx_v)
        for j in range(_CH // nl):
            sl = pl.ds(j * nl, nl)
            idx_v[sl] = idx_v[sl] + off_vec
        pltpu.async_copy(left_hbm.at[idx_v], rows_v, sem).wait()
        pltpu.sync_copy(rows_v, out_hbm.at[pl.ds(start, _CH)])


def _sc_gather(left2, idx):
    mesh = plsc.VectorSubcoreMesh(core_axis_name="c", subcore_axis_name="s")
    f = functools.partial(
        pl.kernel,
        mesh=mesh,
        out_type=jax.ShapeDtypeStruct((_BS, _D), jnp.float32),
        scratch_types=[
            pltpu.VMEM((_CH,), jnp.int32),
            pltpu.VMEM((_CH, _D), jnp.float32),
            pltpu.SemaphoreType.DMA,
        ],
    )(_sc_gather_body)
    return f(left2, idx)


def _wfuse_body(w1t_ref, wvt_ref, b1c_ref, bv_ref, wvf_ref, bvf_ref):
    wf = jnp.dot(w1t_ref[...], wvt_ref[...], preferred_element_type=jnp.float32)
    wvf_ref[...] = wf.astype(jnp.bfloat16)
    t = wvt_ref[...].astype(jnp.float32) * b1c_ref[...]
    bvf_ref[...] = jnp.sum(t, axis=0, keepdims=True) + bv_ref[...]


def _wfuse(w1t_b, wvt_b, b1c, bv_row):
    return pl.pallas_call(
        _wfuse_body,
        out_shape=(
            jax.ShapeDtypeStruct((_D, _D), jnp.bfloat16),
            jax.ShapeDtypeStruct((1, _D), jnp.float32),
        ),
    )(w1t_b, wvt_b, b1c, bv_row)


def _main_body(state_ref, g_ref, lng_ref, lnb_ref, wq_ref, bq_ref,
               wk_ref, bk_ref, wvf_ref, bvf_ref, wo_ref, bo_ref,
               w2_ref, b2_ref, out_ref):
    x = state_ref[...]
    m = jnp.mean(x, axis=1, keepdims=True)
    xc = x - m
    var = jnp.mean(xc * xc, axis=1, keepdims=True)
    xn = lng_ref[...] * (xc * lax.rsqrt(var + 1e-6)) + lnb_ref[...]
    xnb = xn.astype(jnp.bfloat16)
    q = jnp.dot(xnb, wq_ref[...], preferred_element_type=jnp.float32) + bq_ref[...]
    k = jnp.dot(xnb, wk_ref[...], preferred_element_type=jnp.float32) + bk_ref[...]
    v = jnp.dot(g_ref[...].astype(jnp.bfloat16), wvf_ref[...],
                preferred_element_type=jnp.float32) + bvf_ref[...]
    q3 = q.reshape(_R, _H, _DK)
    k3 = k.reshape(_R, _H, _DK)
    v3 = v.reshape(_R, _H, _DK)
    s1 = jnp.sum(q3 * k3, axis=-1, keepdims=True) / _SCALE
    s2 = jnp.sum(q3 * v3, axis=-1, keepdims=True) / _SCALE
    d = s1 - s2
    e = jnp.exp(-jnp.abs(d))
    p0 = jnp.where(d >= 0, 1.0 / (1.0 + e), e / (1.0 + e))
    o = (p0 * k3 + (1.0 - p0) * v3).reshape(_R, _D)
    comb = jnp.dot(o.astype(jnp.bfloat16), wo_ref[...],
                   preferred_element_type=jnp.float32) + bo_ref[...]
    res = x + comb
    out_ref[...] = jnp.dot(res.astype(jnp.bfloat16), w2_ref[...],
                           preferred_element_type=jnp.float32) + b2_ref[...]


def _main(state2, g2, lng, lnb, wqt, bq_r, wkt, bk_r, wvf, bvf, wot, bo_r, w2t, b2_r):
    row_blk = pl.BlockSpec((_R, _D), lambda i: (i, 0))
    full_w = pl.BlockSpec((_D, _D), lambda i: (0, 0))
    vec = pl.BlockSpec((1, _D), lambda i: (0, 0))
    return pl.pallas_call(
        _main_body,
        grid=(_BS // _R,),
        in_specs=[row_blk, row_blk, vec, vec, full_w, vec, full_w, vec,
                  full_w, vec, full_w, vec, full_w, vec],
        out_specs=row_blk,
        out_shape=jax.ShapeDtypeStruct((_BS, _D), jnp.float32),
        compiler_params=pltpu.CompilerParams(
            dimension_semantics=("parallel",)),
    )(state2, g2, lng, lnb, wqt, bq_r, wkt, bk_r, wvf, bvf, wot, bo_r, w2t, b2_r)


def kernel(state, left, inputad, W1, b1, W2, b2, ln_g, ln_b,
           Wq, bq, Wk, bk, Wv, bv, Wo, bo):
    state2 = state.reshape(_BS, _D)
    left2 = left.reshape(_BS, _D)
    idx = inputad.reshape(_BS).astype(jnp.int32)
    bf = jnp.bfloat16
    wvf, bvf = _wfuse(W1.T.astype(bf), Wv.T.astype(bf),
                      b1.reshape(_D, 1), bv.reshape(1, _D))
    g2 = _sc_gather(left2, idx)
    out2 = _main(state2, g2, ln_g.reshape(1, _D), ln_b.reshape(1, _D),
                 Wq.T.astype(bf), bq.reshape(1, _D),
                 Wk.T.astype(bf), bk.reshape(1, _D),
                 wvf, bvf,
                 Wo.T.astype(bf), bo.reshape(1, _D),
                 W2.T.astype(bf), b2.reshape(1, _D))
    return out2.reshape(_B, _S, _D)
